# Initial kernel scaffold; baseline (speedup 1.0000x reference)
import jax
import jax.numpy as jnp
from jax.experimental import pallas as pl


def kernel(mem, idx, val, W_ih, W_hh, b_ih, b_hh):
    raise NotImplementedError("write your pallas kernel here")


# speedup vs baseline: 1.8779x; 1.8779x over previous
import functools

import jax
import jax.numpy as jnp
from jax import lax
from jax.experimental import pallas as pl
from jax.experimental.pallas import tpu as pltpu
from jax.experimental.pallas import tpu_sc as plsc

M = 100000
D = 128
B = 16384

NC = 2
NS = 16
NW = NC * NS
PER_W = B // NW
CHUNK = 128
NCHUNK = PER_W // CHUNK

_mesh = plsc.VectorSubcoreMesh(
    core_axis_name="c", subcore_axis_name="s", num_cores=NC, num_subcores=NS)
_sc_params = pltpu.CompilerParams(needs_layout_passes=False)


def _wid():
  return lax.axis_index("s") * NC + lax.axis_index("c")


@functools.partial(
    pl.kernel,
    out_type=jax.ShapeDtypeStruct((B, D), jnp.float32),
    mesh=_mesh,
    compiler_params=_sc_params,
    scratch_types=[
        pltpu.VMEM((NCHUNK, CHUNK), jnp.int32),
        pltpu.VMEM((PER_W, D), jnp.float32),
        pltpu.SemaphoreType.DMA,
    ],
)
def _sc_gather(mem_hbm, idx_hbm, h_hbm, idx_v, rows_v, sem):
  w = _wid()
  pltpu.sync_copy(idx_hbm.at[w], idx_v)
  copies = []
  for j in range(NCHUNK):
    copies.append(pltpu.async_copy(
        mem_hbm.at[idx_v.at[j]], rows_v.at[pl.ds(j * CHUNK, CHUNK)], sem))
  for c in copies:
    c.wait()
  pltpu.sync_copy(rows_v, h_hbm.at[pl.ds(w * PER_W, PER_W)])


@functools.partial(
    pl.kernel,
    out_type=jax.ShapeDtypeStruct((B,), jnp.int32),
    mesh=_mesh,
    compiler_params=_sc_params,
    scratch_types=[
        pltpu.VMEM((M,), jnp.int32),
        pltpu.VMEM((B,), jnp.int32),
    ],
)
def _sc_winners(idx_hbm, w_hbm, aux, buf):
  w = _wid()

  @pl.when(w == 0)
  def _():
    pltpu.sync_copy(idx_hbm, buf)
    iota16 = lax.iota(jnp.int32, 16)

    def phase_a(c, carry):
      idx_c = buf[pl.ds(c * 16, 16)]
      pos = c * 16 + iota16
      _counts, last = plsc.scan_count(idx_c)
      plsc.store_scatter(aux, [idx_c], pos, mask=last)
      return carry

    lax.fori_loop(0, B // 16, phase_a, None)

    def phase_b(c, _):
      idx_c = buf[pl.ds(c * 16, 16)]
      buf[pl.ds(c * 16, 16)] = plsc.load_gather(aux, [idx_c])
      return _

    lax.fori_loop(0, B // 16, phase_b, None)
    pltpu.sync_copy(buf, w_hbm)


_BLK = 1024


def _gru_body(val_ref, h_ref, wt_ref, ut_ref, bih_ref, bhh_ref, out_ref):
  v = val_ref[...]
  h = h_ref[...]
  gi = jnp.dot(v, wt_ref[...], preferred_element_type=jnp.float32) + bih_ref[...]
  gh = jnp.dot(h, ut_ref[...], preferred_element_type=jnp.float32) + bhh_ref[...]
  i_r = gi[:, :D]
  i_z = gi[:, D:2 * D]
  i_n = gi[:, 2 * D:]
  h_r = gh[:, :D]
  h_z = gh[:, D:2 * D]
  h_n = gh[:, 2 * D:]
  r = jax.nn.sigmoid(i_r + h_r)
  z = jax.nn.sigmoid(i_z + h_z)
  n = jnp.tanh(i_n + r * h_n)
  out_ref[...] = (1.0 - z) * n + z * h


def _tc_gru(val, h, wt, ut, bih, bhh):
  return pl.pallas_call(
      _gru_body,
      grid=(B // _BLK,),
      in_specs=[
          pl.BlockSpec((_BLK, D), lambda i: (i, 0)),
          pl.BlockSpec((_BLK, D), lambda i: (i, 0)),
          pl.BlockSpec((D, 3 * D), lambda i: (0, 0)),
          pl.BlockSpec((D, 3 * D), lambda i: (0, 0)),
          pl.BlockSpec((1, 3 * D), lambda i: (0, 0)),
          pl.BlockSpec((1, 3 * D), lambda i: (0, 0)),
      ],
      out_specs=pl.BlockSpec((_BLK, D), lambda i: (i, 0)),
      out_shape=jax.ShapeDtypeStruct((B, D), jnp.float32),
  )(val, h, wt, ut, bih, bhh)


@functools.partial(
    pl.kernel,
    out_type=(),
    mesh=_mesh,
    compiler_params=_sc_params,
    scratch_types=[
        pltpu.VMEM((NCHUNK, CHUNK), jnp.int32),
        pltpu.VMEM((NCHUNK, CHUNK), jnp.int32),
        pltpu.VMEM((PER_W, D), jnp.float32),
        pltpu.SemaphoreType.DMA,
    ],
)
def _sc_scatter(hnew_hbm, idx_hbm, win_hbm, out_hbm, idx_v, win_v, rows_v, sem):
  w = _wid()
  pltpu.sync_copy(idx_hbm.at[w], idx_v)
  pltpu.sync_copy(win_hbm.at[w], win_v)
  copies = []
  for j in range(NCHUNK):
    copies.append(pltpu.async_copy(
        hnew_hbm.at[win_v.at[j]], rows_v.at[pl.ds(j * CHUNK, CHUNK)], sem))
  for c in copies:
    c.wait()
  copies = []
  for j in range(NCHUNK):
    copies.append(pltpu.async_copy(
        rows_v.at[pl.ds(j * CHUNK, CHUNK)], out_hbm.at[idx_v.at[j]], sem))
  for c in copies:
    c.wait()


def kernel(mem, idx, val, W_ih, W_hh, b_ih, b_hh):
  idx = idx.astype(jnp.int32)
  idx3 = idx.reshape(NW, NCHUNK, CHUNK)

  h = _sc_gather(mem, idx3)
  win = _sc_winners(idx)

  wt = W_ih.T
  ut = W_hh.T
  bih = b_ih.reshape(1, 3 * D)
  bhh = b_hh.reshape(1, 3 * D)
  h_new = _tc_gru(val, h, wt, ut, bih, bhh)

  out_ref = jax.new_ref(mem)
  _sc_scatter(h_new, idx3, win.reshape(NW, NCHUNK, CHUNK), out_ref)
  return out_ref[...]
